# baseline (speedup 1.0000x reference)
import functools

import jax
import jax.numpy as jnp
from jax import lax
from jax.experimental import pallas as pl
from jax.experimental.pallas import tpu as pltpu
from jax.experimental.pallas import tpu_sc as plsc

_K = 8192
_D = 256
_B = 8192
_M = 2048
_KC = 512

_NC = 2
_NS = 16
_NW = _NC * _NS
_BPW = _B // _NW


def _dist_argmin_body(z_ref, cb_ref, idx_ref, csq_scr):
    @pl.when(pl.program_id(0) == 0)
    def _():
        cbf = cb_ref[...]
        csq_scr[...] = jnp.sum(cbf * cbf, axis=1).reshape(1, _K)

    z = z_ref[...]
    zm2 = z * jnp.float32(-2.0)
    zsq = jnp.sum(z * z, axis=1, keepdims=True)
    minv = jnp.full((_M,), jnp.inf, jnp.float32)
    mini = jnp.zeros((_M,), jnp.float32)
    for c in range(_K // _KC):
        cb = cb_ref[pl.ds(c * _KC, _KC), :]
        csq = csq_scr[0, pl.ds(c * _KC, _KC)].reshape(1, _KC)
        sc2 = lax.dot_general(zm2, cb, (((1,), (1,)), ((), ())),
                              preferred_element_type=jnp.float32)
        d = (zsq + csq) + sc2
        lmin = jnp.min(d, axis=1, keepdims=True)
        iota = lax.broadcasted_iota(jnp.int32, (_M, _KC), 1).astype(jnp.float32)
        lidx = jnp.min(jnp.where(d == lmin, iota, jnp.float32(2**30)), axis=1)
        lminv = lmin.reshape(_M)
        take = lminv < minv
        mini = jnp.where(take, lidx + jnp.float32(c * _KC), mini)
        minv = jnp.where(take, lminv, minv)
    idx_ref[...] = mini.astype(jnp.int32)


_dist_argmin = pl.pallas_call(
    _dist_argmin_body,
    grid=(_B // _M,),
    in_specs=[
        pl.BlockSpec((_M, _D), lambda t: (t, 0)),
        pl.BlockSpec((_K, _D), lambda t: (0, 0)),
    ],
    out_specs=pl.BlockSpec((_M,), lambda t: (t,)),
    out_shape=jax.ShapeDtypeStruct((_B,), jnp.int32),
    scratch_shapes=[pltpu.VMEM((1, _K), jnp.float32)],
)


def _sc_gather_body(cb_hbm, idx_hbm, out_hbm, idx_v, rows_v, sem):
    wid = lax.axis_index("s") * _NC + lax.axis_index("c")
    base = wid * _BPW
    pltpu.sync_copy(idx_hbm.at[pl.ds(base, _BPW)], idx_v)
    pltpu.async_copy(cb_hbm.at[idx_v], rows_v, sem).wait()
    pltpu.sync_copy(rows_v, out_hbm.at[pl.ds(base, _BPW)])


@functools.cache
def _sc_gather():
    return pl.kernel(
        _sc_gather_body,
        out_type=jax.ShapeDtypeStruct((_B, _D), jnp.float32),
        mesh=plsc.VectorSubcoreMesh(core_axis_name="c", subcore_axis_name="s"),
        scratch_types=[
            pltpu.VMEM((_BPW,), jnp.int32),
            pltpu.VMEM((_BPW, _D), jnp.float32),
            pltpu.SemaphoreType.DMA,
        ],
    )


def _st_loss_body(z_ref, q_ref, out_ref, loss_ref, acc_ref):
    t = pl.program_id(0)
    z = z_ref[...]
    diff = q_ref[...] - z
    out_ref[...] = z + diff
    s = jnp.sum(diff * diff)

    @pl.when(t == 0)
    def _():
        acc_ref[0] = s

    @pl.when(t > 0)
    def _():
        acc_ref[0] = acc_ref[0] + s

    @pl.when(t == pl.num_programs(0) - 1)
    def _():
        loss_ref[0] = acc_ref[0]


_st_loss = pl.pallas_call(
    _st_loss_body,
    grid=(8,),
    in_specs=[
        pl.BlockSpec((_B // 8, _D), lambda t: (t, 0)),
        pl.BlockSpec((_B // 8, _D), lambda t: (t, 0)),
    ],
    out_specs=[
        pl.BlockSpec((_B // 8, _D), lambda t: (t, 0)),
        pl.BlockSpec(memory_space=pltpu.SMEM),
    ],
    out_shape=[
        jax.ShapeDtypeStruct((_B, _D), jnp.float32),
        jax.ShapeDtypeStruct((1,), jnp.float32),
    ],
    scratch_shapes=[pltpu.SMEM((1,), jnp.float32)],
)


def kernel(z, codebook):
    zf = z.reshape(_B, _D)
    idx = _dist_argmin(zf, codebook)
    zq = _sc_gather()(codebook, idx)
    zq_out, ssq = _st_loss(zf, zq)
    m = ssq[0] / jnp.float32(_B * _D)
    loss = m + 1.0 * m
    return zq_out.reshape(z.shape), idx.reshape(z.shape[:2]), loss


# speedup vs baseline: 1.0133x; 1.0133x over previous
import functools

import jax
import jax.numpy as jnp
from jax import lax
from jax.experimental import pallas as pl
from jax.experimental.pallas import tpu as pltpu
from jax.experimental.pallas import tpu_sc as plsc

_K = 8192
_D = 256
_B = 8192
_M = 2048
_KC = 1024

_NC = 2
_NS = 16
_NW = _NC * _NS
_BPW = _B // _NW


def _dist_argmin_body(z_ref, cb_ref, idx_ref, csq_scr):
    @pl.when(pl.program_id(0) == 0)
    def _():
        cbf = cb_ref[...]
        csq_scr[...] = jnp.sum(cbf * cbf, axis=1).reshape(1, _K)

    z = z_ref[...]
    zm2 = z * jnp.float32(-2.0)
    zsq = jnp.sum(z * z, axis=1, keepdims=True)
    minv = jnp.full((_M,), jnp.inf, jnp.float32)
    mini = jnp.zeros((_M,), jnp.float32)
    for c in range(_K // _KC):
        cb = cb_ref[pl.ds(c * _KC, _KC), :]
        csq = csq_scr[0, pl.ds(c * _KC, _KC)].reshape(1, _KC)
        sc2 = lax.dot_general(zm2, cb, (((1,), (1,)), ((), ())),
                              preferred_element_type=jnp.float32)
        d = (zsq + csq) + sc2
        lmin = jnp.min(d, axis=1, keepdims=True)
        iota = lax.broadcasted_iota(jnp.int32, (_M, _KC), 1).astype(jnp.float32)
        lidx = jnp.min(jnp.where(d == lmin, iota, jnp.float32(2**30)), axis=1)
        lminv = lmin.reshape(_M)
        take = lminv < minv
        mini = jnp.where(take, lidx + jnp.float32(c * _KC), mini)
        minv = jnp.where(take, lminv, minv)
    idx_ref[...] = mini.astype(jnp.int32)


_dist_argmin = pl.pallas_call(
    _dist_argmin_body,
    grid=(_B // _M,),
    in_specs=[
        pl.BlockSpec((_M, _D), lambda t: (t, 0)),
        pl.BlockSpec((_K, _D), lambda t: (0, 0)),
    ],
    out_specs=pl.BlockSpec((_M,), lambda t: (t,)),
    out_shape=jax.ShapeDtypeStruct((_B,), jnp.int32),
    scratch_shapes=[pltpu.VMEM((1, _K), jnp.float32)],
)


def _sc_gather_body(cb_hbm, idx_hbm, out_hbm, idx_v, rows_v, sem):
    wid = lax.axis_index("s") * _NC + lax.axis_index("c")
    base = wid * _BPW
    pltpu.sync_copy(idx_hbm.at[pl.ds(base, _BPW)], idx_v)
    pltpu.async_copy(cb_hbm.at[idx_v], rows_v, sem).wait()
    pltpu.sync_copy(rows_v, out_hbm.at[pl.ds(base, _BPW)])


@functools.cache
def _sc_gather():
    return pl.kernel(
        _sc_gather_body,
        out_type=jax.ShapeDtypeStruct((_B, _D), jnp.float32),
        mesh=plsc.VectorSubcoreMesh(core_axis_name="c", subcore_axis_name="s"),
        scratch_types=[
            pltpu.VMEM((_BPW,), jnp.int32),
            pltpu.VMEM((_BPW, _D), jnp.float32),
            pltpu.SemaphoreType.DMA,
        ],
    )


def _st_loss_body(z_ref, q_ref, out_ref, loss_ref, acc_ref):
    t = pl.program_id(0)
    z = z_ref[...]
    diff = q_ref[...] - z
    out_ref[...] = z + diff
    s = jnp.sum(diff * diff)

    @pl.when(t == 0)
    def _():
        acc_ref[0] = s

    @pl.when(t > 0)
    def _():
        acc_ref[0] = acc_ref[0] + s

    @pl.when(t == pl.num_programs(0) - 1)
    def _():
        loss_ref[0] = acc_ref[0]


_st_loss = pl.pallas_call(
    _st_loss_body,
    grid=(8,),
    in_specs=[
        pl.BlockSpec((_B // 8, _D), lambda t: (t, 0)),
        pl.BlockSpec((_B // 8, _D), lambda t: (t, 0)),
    ],
    out_specs=[
        pl.BlockSpec((_B // 8, _D), lambda t: (t, 0)),
        pl.BlockSpec(memory_space=pltpu.SMEM),
    ],
    out_shape=[
        jax.ShapeDtypeStruct((_B, _D), jnp.float32),
        jax.ShapeDtypeStruct((1,), jnp.float32),
    ],
    scratch_shapes=[pltpu.SMEM((1,), jnp.float32)],
)


def kernel(z, codebook):
    zf = z.reshape(_B, _D)
    idx = _dist_argmin(zf, codebook)
    zq = _sc_gather()(codebook, idx)
    zq_out, ssq = _st_loss(zf, zq)
    m = ssq[0] / jnp.float32(_B * _D)
    loss = m + 1.0 * m
    return zq_out.reshape(z.shape), idx.reshape(z.shape[:2]), loss
